# baseline (speedup 1.0000x reference)
import functools

import jax
import jax.numpy as jnp
from jax.experimental import pallas as pl
from jax.experimental.pallas import tpu as pltpu

_TOPK = 5
_NEG = float("-inf")


def _fused_body(q_ref, l_ref, dist_ref, idx_ref, rv_ref, ri_ref, *,
                kt, k_total, nsteps, topk):
    t = pl.program_id(0)
    b = q_ref.shape[0]

    @pl.when(t == 0)
    def _init():
        rv_ref[...] = jnp.full((b, 128), _NEG, jnp.float32)
        ri_ref[...] = jnp.zeros((b, 128), jnp.int32)

    q = q_ref[...]
    lab = l_ref[...]
    dots = jax.lax.dot_general(q, lab, (((1,), (1,)), ((), ())),
                               preferred_element_type=jnp.float32)
    qn = jnp.sqrt(jnp.sum(q * q, axis=1, keepdims=True))
    ones8 = jnp.ones((8, q.shape[1]), jnp.float32)
    lsq = jax.lax.dot_general(ones8, lab * lab, (((1,), (1,)), ((), ())),
                              precision=jax.lax.Precision.HIGHEST,
                              preferred_element_type=jnp.float32)
    ln = jnp.sqrt(lsq[0:1, :])
    denom = jnp.maximum(qn * ln, 1e-8)
    dist = 1.0 - dots / denom
    dist_ref[...] = dist

    col = jax.lax.broadcasted_iota(jnp.int32, (b, kt), 1)
    gid0 = t * kt
    s = jnp.where(col + gid0 < k_total, -dist, _NEG)

    bv, bi = [], []
    for _ in range(topk):
        m = jnp.max(s, axis=1, keepdims=True)
        c = jnp.min(jnp.where(s == m, col, kt), axis=1, keepdims=True)
        bv.append(m)
        bi.append(c + gid0)
        s = jnp.where(col == c, _NEG, s)

    colw = jax.lax.broadcasted_iota(jnp.int32, (b, 128), 1)
    cv = rv_ref[...]
    ci = ri_ref[...]
    for j in range(topk):
        cv = jnp.where(colw == topk + j, bv[j], cv)
        ci = jnp.where(colw == topk + j, bi[j], ci)
    nv = jnp.full((b, 128), _NEG, jnp.float32)
    ni = jnp.zeros((b, 128), jnp.int32)
    for j in range(topk):
        m = jnp.max(cv, axis=1, keepdims=True)
        c = jnp.min(jnp.where(cv == m, colw, 128), axis=1, keepdims=True)
        sel = colw == c
        gi = jnp.max(jnp.where(sel, ci, -1), axis=1, keepdims=True)
        nv = jnp.where(colw == j, m, nv)
        ni = jnp.where(colw == j, gi, ni)
        cv = jnp.where(sel, _NEG, cv)
    rv_ref[...] = nv
    ri_ref[...] = ni

    @pl.when(t == nsteps - 1)
    def _fin():
        idx_ref[...] = ri_ref[:, :topk]


def kernel(queries, labelvec, k):
    del k
    b, d = queries.shape
    k_total = labelvec.shape[0]
    kt = 512
    nsteps = pl.cdiv(k_total, kt)
    dist, idx = pl.pallas_call(
        functools.partial(_fused_body, kt=kt, k_total=k_total,
                          nsteps=nsteps, topk=_TOPK),
        grid=(nsteps,),
        in_specs=[pl.BlockSpec((b, d), lambda t: (0, 0)),
                  pl.BlockSpec((kt, d), lambda t: (t, 0))],
        out_specs=[pl.BlockSpec((b, kt), lambda t: (0, t)),
                   pl.BlockSpec((b, _TOPK), lambda t: (0, 0))],
        out_shape=[jax.ShapeDtypeStruct((b, k_total), jnp.float32),
                   jax.ShapeDtypeStruct((b, _TOPK), jnp.int32)],
        scratch_shapes=[pltpu.VMEM((b, 128), jnp.float32),
                        pltpu.VMEM((b, 128), jnp.int32)],
        compiler_params=pltpu.CompilerParams(
            dimension_semantics=("arbitrary",)),
    )(queries, labelvec)
    return dist, idx


# speedup vs baseline: 2.7087x; 2.7087x over previous
import functools

import jax
import jax.numpy as jnp
from jax import lax
from jax.experimental import pallas as pl
from jax.experimental.pallas import tpu as pltpu
from jax.experimental.pallas import tpu_sc as plsc

_TOPK = 5
_INF = float("inf")
_BIG = 1 << 30
_CHUNK = 128
_SLOT = 8
_NC, _NS, _L = 2, 16, 16


def _tc_body(q_ref, l_ref, dist_ref, cmin_ref, *, kt, k_total):
    t = pl.program_id(0)
    b = q_ref.shape[0]

    q = q_ref[...]
    lab = l_ref[...]
    dots = jax.lax.dot_general(q, lab, (((1,), (1,)), ((), ())),
                               preferred_element_type=jnp.float32)
    qn = jnp.sqrt(jnp.sum(q * q, axis=1, keepdims=True))
    ones8 = jnp.ones((8, q.shape[1]), jnp.float32)
    lsq = jax.lax.dot_general(ones8, lab * lab, (((1,), (1,)), ((), ())),
                              precision=jax.lax.Precision.HIGHEST,
                              preferred_element_type=jnp.float32)
    ln = jnp.sqrt(lsq[0:1, :])
    denom = jnp.maximum(qn * ln, 1e-8)
    dist = 1.0 - dots / denom
    dist_ref[...] = dist

    col = jax.lax.broadcasted_iota(jnp.int32, (b, kt), 1)
    dm = jnp.where(col + t * kt < k_total, dist, _INF)
    for c in range(kt // _CHUNK):
        cm = jnp.min(dm[:, c * _CHUNK:(c + 1) * _CHUNK], axis=1,
                     keepdims=True)
        cmin_ref[0, :, c:c + 1] = cm


def _make_sc_kernel(b, k_total, nsteps, cpb, rows_per_w):
    nchunks = (k_total + _CHUNK - 1) // _CHUNK
    nslots = nsteps * _SLOT
    nvec = (nslots + _L - 1) // _L
    vpc = _CHUNK // _L

    mesh = plsc.VectorSubcoreMesh(core_axis_name="c", subcore_axis_name="s")

    @functools.partial(
        pl.kernel, mesh=mesh,
        compiler_params=pltpu.CompilerParams(needs_layout_passes=False),
        out_type=jax.ShapeDtypeStruct((b, _L), jnp.int32),
        scratch_types=[
            pltpu.VMEM((rows_per_w, nslots), jnp.float32),
            pltpu.VMEM((_TOPK, 8, _CHUNK), jnp.float32),
            pltpu.VMEM((rows_per_w, _L), jnp.int32),
        ],
    )
    def sc_kernel(cmin_hbm, dist_hbm, out_hbm, cmv, dbuf, resv):
        wid = lax.axis_index("s") * _NC + lax.axis_index("c")
        base_row = wid * rows_per_w
        iota = lax.iota(jnp.int32, _L)
        lane0 = iota == 0
        inf_v = jnp.full((_L,), _INF, jnp.float32)

        pltpu.sync_copy(cmin_hbm.at[pl.ds(base_row, rows_per_w), :], cmv)

        def row_fn(i, carry):
            row_v = jnp.full((_L,), i, jnp.int32)

            ids = jnp.full((_L,), nchunks, jnp.int32)
            for j in range(_TOPK):
                def cscan(w, sc):
                    acc, gcc = sc
                    p = iota + w * _L
                    d = cmv[i, pl.ds(w * _L, _L)]
                    g = ((p >> 3) << 2) + (p & 7)
                    ok = ((p & 7) < cpb) & (g < nchunks)
                    d = jnp.where(ok, d, _INF)
                    lt = d < acc
                    return (jnp.where(lt, d, acc), jnp.where(lt, g, gcc))

                acc, gcc = lax.fori_loop(
                    0, nvec, cscan, (inf_v, jnp.zeros((_L,), jnp.int32)))
                m = jnp.min(acc)
                gj = jnp.min(jnp.where(acc == m, gcc, _BIG))
                ids = jnp.where(iota == j, gj, ids)
                slot = ((gj >> 2) << 3) + (gj & 3)
                plsc.store_scatter(
                    cmv, [row_v, jnp.full((_L,), slot, jnp.int32)],
                    inf_v, mask=lane0)

            ids, _ = plsc.sort_key_val(ids, iota)
            his = jnp.minimum(ids * _CHUNK + _CHUNK, k_total)
            lbs = [jnp.min(jnp.where(iota == j, ids * _CHUNK, _BIG))
                   for j in range(_TOPK)]
            hbs = [jnp.min(jnp.where(iota == j, his, _BIG))
                   for j in range(_TOPK)]
            rb = pl.multiple_of(base_row + ((i >> 3) << 3), 8)
            r8 = i & 7
            for j in range(_TOPK):
                pltpu.sync_copy(
                    dist_hbm.at[pl.ds(rb, 8),
                                pl.ds(pl.multiple_of(lbs[j], _CHUNK),
                                      _CHUNK)],
                    dbuf.at[j])

            res = jnp.zeros((_L,), jnp.int32)
            for j in range(_TOPK):
                carry_v = (inf_v, jnp.zeros((_L,), jnp.int32),
                           jnp.zeros((_L,), jnp.int32))
                for cj in range(_TOPK):
                    def vscan(w2, sc, cj=cj):
                        acc, gcc, pcc = sc
                        d = dbuf[cj, r8, pl.ds(w2 * _L, _L)]
                        g = lbs[cj] + w2 * _L + iota
                        pos = cj * _CHUNK + w2 * _L + iota
                        d = jnp.where(g < hbs[cj], d, _INF)
                        lt = d < acc
                        return (jnp.where(lt, d, acc),
                                jnp.where(lt, g, gcc),
                                jnp.where(lt, pos, pcc))

                    carry_v = lax.fori_loop(0, vpc, vscan, carry_v)
                acc, gcc, pcc = carry_v
                m = jnp.min(acc)
                hit = acc == m
                gj = jnp.min(jnp.where(hit, gcc, _BIG))
                pj = jnp.min(jnp.where(hit & (gcc == gj), pcc, _BIG))
                res = jnp.where(iota == j, gj, res)
                plsc.store_scatter(
                    dbuf, [jnp.full((_L,), pj >> 7, jnp.int32),
                           jnp.full((_L,), r8, jnp.int32),
                           jnp.full((_L,), pj & (_CHUNK - 1), jnp.int32)],
                    inf_v, mask=lane0)

            resv[i, :] = res
            return carry

        lax.fori_loop(0, rows_per_w, row_fn, 0)
        pltpu.sync_copy(resv, out_hbm.at[pl.ds(base_row, rows_per_w)])

    return sc_kernel


def kernel(queries, labelvec, k):
    del k
    b, d = queries.shape
    k_total = labelvec.shape[0]
    kt = 512
    nsteps = pl.cdiv(k_total, kt)
    cpb = kt // _CHUNK
    dist, cmin = pl.pallas_call(
        functools.partial(_tc_body, kt=kt, k_total=k_total),
        grid=(nsteps,),
        in_specs=[pl.BlockSpec((b, d), lambda t: (0, 0)),
                  pl.BlockSpec((kt, d), lambda t: (t, 0))],
        out_specs=[pl.BlockSpec((b, kt), lambda t: (0, t)),
                   pl.BlockSpec((1, b, _SLOT), lambda t: (t, 0, 0))],
        out_shape=[jax.ShapeDtypeStruct((b, k_total), jnp.float32),
                   jax.ShapeDtypeStruct((nsteps, b, _SLOT), jnp.float32)],
        compiler_params=pltpu.CompilerParams(
            dimension_semantics=("arbitrary",)),
    )(queries, labelvec)

    rows_per_w = b // (_NC * _NS)
    cmin2 = cmin.transpose(1, 0, 2).reshape(b, nsteps * _SLOT)
    sc = _make_sc_kernel(b, k_total, nsteps, cpb, rows_per_w)
    idxpad = sc(cmin2, dist)
    return dist, idxpad[:, :_TOPK]
